# baseline (speedup 1.0000x reference)
import functools
import math

import jax
import jax.numpy as jnp
from jax import lax
from jax.experimental import pallas as pl
from jax.experimental.pallas import tpu as pltpu
from jax.experimental.pallas import tpu_sc as plsc

_TOPK = 2
_NC = 2
_NS = 16
_NW = _NC * _NS


def _router_body(x_ref, g_ref, sd0_ref, sd1_ref, sc0_ref, sc1_ref, w0_ref,
                 w1_ref, counts_ref, loss_ref, base_ref, psum_ref, *,
                 capacity, e_num, n_chunks, chunk, trash):
    c = pl.program_id(0)

    @pl.when(c == 0)
    def _():
        base_ref[...] = jnp.zeros_like(base_ref)
        psum_ref[...] = jnp.zeros_like(psum_ref)

    x = x_ref[...]
    g = g_ref[...]
    logits = lax.dot_general(x, g, (((1,), (1,)), ((), ())),
                             preferred_element_type=jnp.float32)
    m = jnp.max(logits, axis=1, keepdims=True)
    ex = jnp.exp(logits - m)
    p = ex / jnp.sum(ex, axis=1, keepdims=True)

    iota = lax.broadcasted_iota(jnp.int32, (chunk, e_num), 1)
    v1 = jnp.max(p, axis=1, keepdims=True)
    i1 = jnp.min(jnp.where(p == v1, iota, e_num), axis=1, keepdims=True)
    pm = jnp.where(iota == i1, -jnp.inf, p)
    v2 = jnp.max(pm, axis=1, keepdims=True)
    i2 = jnp.min(jnp.where(pm == v2, iota, e_num), axis=1, keepdims=True)
    s = v1 + v2 + 1e-9
    w0 = v1 / s
    w1 = v2 / s

    cmat = (iota == i1).astype(jnp.float32) + (iota == i2).astype(jnp.float32)

    ri = lax.broadcasted_iota(jnp.int32, (chunk, chunk), 0)
    ci = lax.broadcasted_iota(jnp.int32, (chunk, chunk), 1)
    ltri = (ci < ri).astype(jnp.float32)
    pref = base_ref[...] + lax.dot_general(
        ltri, cmat, (((1,), (0,)), ((), ())),
        preferred_element_type=jnp.float32)

    l0 = jnp.sum(jnp.where(iota == i1, pref, 0.0), axis=1,
                 keepdims=True).astype(jnp.int32)
    l1 = jnp.sum(jnp.where(iota == i2, pref, 0.0), axis=1,
                 keepdims=True).astype(jnp.int32)
    keep0 = l0 < capacity
    keep1 = l1 < capacity
    slot0 = i1 * capacity + l0
    slot1 = i2 * capacity + l1
    sd0_ref[...] = jnp.where(keep0, slot0, trash)
    sd1_ref[...] = jnp.where(keep1, slot1, trash)
    sc0_ref[...] = jnp.where(keep0, slot0, 0)
    sc1_ref[...] = jnp.where(keep1, slot1, 0)
    w0_ref[...] = jnp.where(keep0, w0, 0.0)
    w1_ref[...] = jnp.where(keep1, w1, 0.0)

    newbase = base_ref[...] + jnp.sum(cmat, axis=0, keepdims=True)
    base_ref[...] = newbase
    newpsum = psum_ref[...] + jnp.sum(p, axis=0, keepdims=True)
    psum_ref[...] = newpsum
    counts_ref[...] = newbase

    @pl.when(c == n_chunks - 1)
    def _():
        n_tok = n_chunks * chunk
        pmean = newpsum / float(n_tok)
        f = newbase / float(n_tok * _TOPK)
        loss_ref[0, 0] = jnp.sum(pmean * f) * float(e_num)


def _ffn_body(counts_ref, bufx_ref, w1_ref, wg_ref, w2_ref, out_ref, xb_ref,
              *, nh, rt, capacity):
    e = pl.program_id(0)
    h = pl.program_id(1)
    cnt = counts_ref[0, e].astype(jnp.int32)

    @pl.when(h == 0)
    def _():
        xb_ref[...] = bufx_ref[...].astype(jnp.bfloat16)

    w1b = w1_ref[0].astype(jnp.bfloat16)
    wgb = wg_ref[0].astype(jnp.bfloat16)
    w2b = w2_ref[0].astype(jnp.bfloat16)

    for i in range(capacity // rt):
        rs = pl.ds(i * rt, rt)

        @pl.when(i * rt < cnt)
        def _(i=i, rs=rs):
            x = xb_ref[rs, :]
            a = lax.dot_general(x, w1b, (((1,), (1,)), ((), ())),
                                preferred_element_type=jnp.float32)
            g = lax.dot_general(x, wgb, (((1,), (1,)), ((), ())),
                                preferred_element_type=jnp.float32)
            act = (a * jax.nn.sigmoid(a) * g).astype(jnp.bfloat16)
            delta = lax.dot_general(act, w2b, (((1,), (1,)), ((), ())),
                                    preferred_element_type=jnp.float32)

            @pl.when(h == 0)
            def _():
                out_ref[rs, :] = delta

            @pl.when(jnp.logical_and(h > 0, h < nh - 1))
            def _():
                out_ref[rs, :] = out_ref[rs, :] + delta

            @pl.when(h == nh - 1)
            def _(i=i):
                rows = i * rt + lax.broadcasted_iota(jnp.int32, (rt, 1), 0)
                acc = out_ref[rs, :] + delta
                out_ref[rs, :] = jnp.where(rows < cnt, acc, 0.0)

        @pl.when(jnp.logical_and(i * rt >= cnt, h == 0))
        def _(rs=rs):
            out_ref[rs, :] = jnp.zeros((rt, out_ref.shape[1]), jnp.float32)


def _dispatch_sc(x2d, sd0r, sd1r, nrows, d):
    mesh = plsc.VectorSubcoreMesh(core_axis_name="c", subcore_axis_name="s",
                                  num_cores=_NC)

    @functools.partial(
        pl.kernel,
        out_type=jax.ShapeDtypeStruct((nrows + 8, d), jnp.float32),
        mesh=mesh,
        scratch_types=[
            pltpu.VMEM((8, 16), jnp.int32),
            pltpu.VMEM((8, 16), jnp.int32),
            pltpu.VMEM((16, d), jnp.float32),
            pltpu.SemaphoreType.DMA,
        ],
    )
    def _dispatch(x_hbm, sd0_hbm, sd1_hbm, bufx_hbm, idx0_v, idx1_v, rows_v,
                  sem):
        w = lax.axis_index("s") * _NC + lax.axis_index("c")
        pltpu.sync_copy(sd0_hbm.at[pl.ds(w * 8, 8)], idx0_v)
        pltpu.sync_copy(sd1_hbm.at[pl.ds(w * 8, 8)], idx1_v)
        for j in range(8):
            pltpu.sync_copy(x_hbm.at[pl.ds(w * 128 + j * 16, 16)], rows_v)
            pltpu.async_copy(rows_v, bufx_hbm.at[idx0_v[j]], sem).wait()
            pltpu.async_copy(rows_v, bufx_hbm.at[idx1_v[j]], sem).wait()

    return _dispatch(x2d, sd0r, sd1r)


def _combine_sc(bufout, sc0r, sc1r, w0r, w1r, n, d):
    mesh = plsc.VectorSubcoreMesh(core_axis_name="c", subcore_axis_name="s",
                                  num_cores=_NC)

    @functools.partial(
        pl.kernel,
        out_type=jax.ShapeDtypeStruct((n, d), jnp.float32),
        mesh=mesh,
        scratch_types=[
            pltpu.VMEM((8, 16), jnp.int32),
            pltpu.VMEM((8, 16), jnp.int32),
            pltpu.VMEM((8, 16), jnp.float32),
            pltpu.VMEM((8, 16), jnp.float32),
            pltpu.VMEM((16, d), jnp.float32),
            pltpu.VMEM((16, d), jnp.float32),
            pltpu.VMEM((16, d), jnp.float32),
            pltpu.SemaphoreType.DMA,
        ],
    )
    def _combine(bufout_hbm, sc0_hbm, sc1_hbm, w0_hbm, w1_hbm, out_hbm,
                 idx0_v, idx1_v, wa_v, wb_v, r0_v, r1_v, o_v, sem):
        w = lax.axis_index("s") * _NC + lax.axis_index("c")
        pltpu.sync_copy(sc0_hbm.at[pl.ds(w * 8, 8)], idx0_v)
        pltpu.sync_copy(sc1_hbm.at[pl.ds(w * 8, 8)], idx1_v)
        pltpu.sync_copy(w0_hbm.at[pl.ds(w * 8, 8)], wa_v)
        pltpu.sync_copy(w1_hbm.at[pl.ds(w * 8, 8)], wb_v)
        def chunk(j, carry):
            pltpu.async_copy(bufout_hbm.at[idx0_v[j]], r0_v, sem).wait()
            pltpu.async_copy(bufout_hbm.at[idx1_v[j]], r1_v, sem).wait()
            wav = wa_v[j]
            wbv = wb_v[j]
            was = [wav[t] for t in range(16)]
            wbs = [wbv[t] for t in range(16)]

            def body(v, c2):
                base = pl.multiple_of(v * 128, 128)
                for t in range(16):
                    wa = was[t]
                    wb = wbs[t]
                    for u in range(8):
                        sl = pl.ds(base + u * 16, 16)
                        o_v[t, sl] = wa * r0_v[t, sl] + wb * r1_v[t, sl]
                return c2

            lax.fori_loop(0, d // 128, body, 0)
            pltpu.sync_copy(o_v, out_hbm.at[pl.ds(w * 128 + j * 16, 16)])
            return carry

        lax.fori_loop(0, 8, chunk, 0)

    return _combine(bufout, sc0r, sc1r, w0r, w1r)


def kernel(x, gate_w, fc1_w, gating_w, fc2_w):
    bs, seq, d = x.shape
    e_num, hd, _ = fc1_w.shape
    n = bs * seq
    nk = n * _TOPK
    avg = (nk + e_num - 1) // e_num
    capacity = max(int(math.ceil(avg * 1.25)), 4)
    trash = e_num * capacity
    nrows = e_num * capacity

    chunk = 256
    n_chunks = n // chunk
    assert n % chunk == 0 and n % (_NW * 128) == 0

    x2d = x.reshape(n, d)

    router = pl.pallas_call(
        functools.partial(_router_body, capacity=capacity, e_num=e_num,
                          n_chunks=n_chunks, chunk=chunk, trash=trash),
        grid=(n_chunks,),
        in_specs=[
            pl.BlockSpec((chunk, d), lambda c: (c, 0)),
            pl.BlockSpec((e_num, d), lambda c: (0, 0)),
        ],
        out_specs=[
            pl.BlockSpec((chunk, 1), lambda c: (c, 0)),
            pl.BlockSpec((chunk, 1), lambda c: (c, 0)),
            pl.BlockSpec((chunk, 1), lambda c: (c, 0)),
            pl.BlockSpec((chunk, 1), lambda c: (c, 0)),
            pl.BlockSpec((chunk, 1), lambda c: (c, 0)),
            pl.BlockSpec((chunk, 1), lambda c: (c, 0)),
            pl.BlockSpec((1, e_num), lambda c: (0, 0)),
            pl.BlockSpec(memory_space=pltpu.SMEM),
        ],
        out_shape=[
            jax.ShapeDtypeStruct((n, 1), jnp.int32),
            jax.ShapeDtypeStruct((n, 1), jnp.int32),
            jax.ShapeDtypeStruct((n, 1), jnp.int32),
            jax.ShapeDtypeStruct((n, 1), jnp.int32),
            jax.ShapeDtypeStruct((n, 1), jnp.float32),
            jax.ShapeDtypeStruct((n, 1), jnp.float32),
            jax.ShapeDtypeStruct((1, e_num), jnp.float32),
            jax.ShapeDtypeStruct((1, 1), jnp.float32),
        ],
        scratch_shapes=[
            pltpu.VMEM((1, e_num), jnp.float32),
            pltpu.VMEM((1, e_num), jnp.float32),
        ],
    )
    sd0, sd1, sc0, sc1, w0, w1, counts, loss = router(x2d, gate_w)

    sd0r = sd0.reshape(n // 16, 16)
    sd1r = sd1.reshape(n // 16, 16)
    sc0r = sc0.reshape(n // 16, 16)
    sc1r = sc1.reshape(n // 16, 16)
    w0r = w0.reshape(n // 16, 16)
    w1r = w1.reshape(n // 16, 16)

    bufx = _dispatch_sc(x2d, sd0r, sd1r, nrows, d)

    hb = 1024
    nh = hd // hb
    ffn = pl.pallas_call(
        functools.partial(_ffn_body, nh=nh, rt=128, capacity=capacity),
        grid=(e_num, nh),
        in_specs=[
            pl.BlockSpec(memory_space=pltpu.SMEM),
            pl.BlockSpec((capacity, d), lambda e, h: (e, 0)),
            pl.BlockSpec((1, hb, d), lambda e, h: (e, h, 0)),
            pl.BlockSpec((1, hb, d), lambda e, h: (e, h, 0)),
            pl.BlockSpec((1, d, hb), lambda e, h: (e, 0, h)),
        ],
        out_specs=pl.BlockSpec((capacity, d), lambda e, h: (e, 0)),
        out_shape=jax.ShapeDtypeStruct((nrows, d), jnp.float32),
        scratch_shapes=[pltpu.VMEM((capacity, d), jnp.bfloat16)],
    )
    bufout = ffn(counts, bufx, fc1_w, gating_w, fc2_w)

    out = _combine_sc(bufout, sc0r, sc1r, w0r, w1r, n, d)

    return out.reshape(bs, seq, d).astype(x.dtype), loss[0, 0]


# speedup vs baseline: 1.8164x; 1.8164x over previous
import functools
import math

import jax
import jax.numpy as jnp
from jax import lax
from jax.experimental import pallas as pl
from jax.experimental.pallas import tpu as pltpu
from jax.experimental.pallas import tpu_sc as plsc

_TOPK = 2
_NC = 2
_NS = 16
_NW = _NC * _NS


def _router_body(x_ref, g_ref, sd0_ref, sd1_ref, sc0_ref, sc1_ref, w0_ref,
                 w1_ref, counts_ref, loss_ref, base_ref, psum_ref, *,
                 capacity, e_num, n_chunks, chunk, trash):
    c = pl.program_id(0)

    @pl.when(c == 0)
    def _():
        base_ref[...] = jnp.zeros_like(base_ref)
        psum_ref[...] = jnp.zeros_like(psum_ref)

    x = x_ref[...]
    g = g_ref[...]
    logits = lax.dot_general(x, g, (((1,), (1,)), ((), ())),
                             preferred_element_type=jnp.float32)
    m = jnp.max(logits, axis=1, keepdims=True)
    ex = jnp.exp(logits - m)
    p = ex / jnp.sum(ex, axis=1, keepdims=True)

    iota = lax.broadcasted_iota(jnp.int32, (chunk, e_num), 1)
    v1 = jnp.max(p, axis=1, keepdims=True)
    i1 = jnp.min(jnp.where(p == v1, iota, e_num), axis=1, keepdims=True)
    pm = jnp.where(iota == i1, -jnp.inf, p)
    v2 = jnp.max(pm, axis=1, keepdims=True)
    i2 = jnp.min(jnp.where(pm == v2, iota, e_num), axis=1, keepdims=True)
    s = v1 + v2 + 1e-9
    w0 = v1 / s
    w1 = v2 / s

    cmat = (iota == i1).astype(jnp.float32) + (iota == i2).astype(jnp.float32)

    ri = lax.broadcasted_iota(jnp.int32, (chunk, chunk), 0)
    ci = lax.broadcasted_iota(jnp.int32, (chunk, chunk), 1)
    ltri = (ci < ri).astype(jnp.float32)
    pref = base_ref[...] + lax.dot_general(
        ltri, cmat, (((1,), (0,)), ((), ())),
        preferred_element_type=jnp.float32)

    l0 = jnp.sum(jnp.where(iota == i1, pref, 0.0), axis=1,
                 keepdims=True).astype(jnp.int32)
    l1 = jnp.sum(jnp.where(iota == i2, pref, 0.0), axis=1,
                 keepdims=True).astype(jnp.int32)
    keep0 = l0 < capacity
    keep1 = l1 < capacity
    slot0 = i1 * capacity + l0
    slot1 = i2 * capacity + l1
    sd0_ref[...] = jnp.where(keep0, slot0, trash)
    sd1_ref[...] = jnp.where(keep1, slot1, trash)
    sc0_ref[...] = jnp.where(keep0, slot0, 0)
    sc1_ref[...] = jnp.where(keep1, slot1, 0)
    w0_ref[...] = jnp.where(keep0, w0, 0.0)
    w1_ref[...] = jnp.where(keep1, w1, 0.0)

    newbase = base_ref[...] + jnp.sum(cmat, axis=0, keepdims=True)
    base_ref[...] = newbase
    newpsum = psum_ref[...] + jnp.sum(p, axis=0, keepdims=True)
    psum_ref[...] = newpsum
    counts_ref[...] = newbase

    @pl.when(c == n_chunks - 1)
    def _():
        n_tok = n_chunks * chunk
        pmean = newpsum / float(n_tok)
        f = newbase / float(n_tok * _TOPK)
        loss_ref[0, 0] = jnp.sum(pmean * f) * float(e_num)


def _ffn_body(counts_ref, bufx_ref, w1_ref, wg_ref, w2_ref, out_ref, xb_ref,
              *, nh, rt, capacity):
    e = pl.program_id(0)
    h = pl.program_id(1)
    cnt = counts_ref[0, e].astype(jnp.int32)

    @pl.when(h == 0)
    def _():
        xb_ref[...] = bufx_ref[...].astype(jnp.bfloat16)

    w1b = w1_ref[0].astype(jnp.bfloat16)
    wgb = wg_ref[0].astype(jnp.bfloat16)
    w2b = w2_ref[0].astype(jnp.bfloat16)

    for i in range(capacity // rt):
        rs = pl.ds(i * rt, rt)

        @pl.when(i * rt < cnt)
        def _(i=i, rs=rs):
            x = xb_ref[rs, :]
            a = lax.dot_general(x, w1b, (((1,), (1,)), ((), ())),
                                preferred_element_type=jnp.float32)
            g = lax.dot_general(x, wgb, (((1,), (1,)), ((), ())),
                                preferred_element_type=jnp.float32)
            act = (a * jax.nn.sigmoid(a) * g).astype(jnp.bfloat16)
            delta = lax.dot_general(act, w2b, (((1,), (1,)), ((), ())),
                                    preferred_element_type=jnp.float32)

            @pl.when(h == 0)
            def _():
                out_ref[rs, :] = delta

            @pl.when(jnp.logical_and(h > 0, h < nh - 1))
            def _():
                out_ref[rs, :] = out_ref[rs, :] + delta

            @pl.when(h == nh - 1)
            def _(i=i):
                rows = i * rt + lax.broadcasted_iota(jnp.int32, (rt, 1), 0)
                acc = out_ref[rs, :] + delta
                out_ref[rs, :] = jnp.where(rows < cnt, acc, 0.0)

        @pl.when(jnp.logical_and(i * rt >= cnt, h == 0))
        def _(rs=rs):
            out_ref[rs, :] = jnp.zeros((rt, out_ref.shape[1]), jnp.float32)


def _dispatch_sc(x2d, sd0r, sd1r, nrows, d):
    mesh = plsc.VectorSubcoreMesh(core_axis_name="c", subcore_axis_name="s",
                                  num_cores=_NC)

    @functools.partial(
        pl.kernel,
        out_type=jax.ShapeDtypeStruct((nrows + 8, d), jnp.float32),
        mesh=mesh,
        scratch_types=[
            pltpu.VMEM((8, 16), jnp.int32),
            pltpu.VMEM((8, 16), jnp.int32),
            pltpu.VMEM((16, d), jnp.float32),
            pltpu.SemaphoreType.DMA,
        ],
    )
    def _dispatch(x_hbm, sd0_hbm, sd1_hbm, bufx_hbm, idx0_v, idx1_v, rows_v,
                  sem):
        w = lax.axis_index("s") * _NC + lax.axis_index("c")
        pltpu.sync_copy(sd0_hbm.at[pl.ds(w * 8, 8)], idx0_v)
        pltpu.sync_copy(sd1_hbm.at[pl.ds(w * 8, 8)], idx1_v)
        for j in range(8):
            pltpu.sync_copy(x_hbm.at[pl.ds(w * 128 + j * 16, 16)], rows_v)
            pltpu.async_copy(rows_v, bufx_hbm.at[idx0_v[j]], sem).wait()
            pltpu.async_copy(rows_v, bufx_hbm.at[idx1_v[j]], sem).wait()

    return _dispatch(x2d, sd0r, sd1r)


def _combine_sc(bufout, sc0r, sc1r, w0r, w1r, n, d):
    mesh = plsc.VectorSubcoreMesh(core_axis_name="c", subcore_axis_name="s",
                                  num_cores=_NC)

    @functools.partial(
        pl.kernel,
        out_type=jax.ShapeDtypeStruct((n, d), jnp.float32),
        mesh=mesh,
        scratch_types=[
            pltpu.VMEM((8, 16), jnp.int32),
            pltpu.VMEM((8, 16), jnp.int32),
            pltpu.VMEM((8, 16), jnp.float32),
            pltpu.VMEM((8, 16), jnp.float32),
            pltpu.VMEM((16, d), jnp.float32),
            pltpu.VMEM((16, d), jnp.float32),
            pltpu.VMEM((16, d), jnp.float32),
            pltpu.SemaphoreType.DMA,
        ],
    )
    def _combine(bufout_hbm, sc0_hbm, sc1_hbm, w0_hbm, w1_hbm, out_hbm,
                 idx0_v, idx1_v, wa_v, wb_v, r0_v, r1_v, o_v, sem):
        w = lax.axis_index("s") * _NC + lax.axis_index("c")
        pltpu.sync_copy(sc0_hbm.at[pl.ds(w * 8, 8)], idx0_v)
        pltpu.sync_copy(sc1_hbm.at[pl.ds(w * 8, 8)], idx1_v)
        pltpu.sync_copy(w0_hbm.at[pl.ds(w * 8, 8)], wa_v)
        pltpu.sync_copy(w1_hbm.at[pl.ds(w * 8, 8)], wb_v)
        def chunk(j, carry):
            pltpu.async_copy(bufout_hbm.at[idx0_v[j]], r0_v, sem).wait()
            pltpu.async_copy(bufout_hbm.at[idx1_v[j]], r1_v, sem).wait()
            wav = wa_v[j]
            wbv = wb_v[j]
            was = [wav[t] for t in range(16)]
            wbs = [wbv[t] for t in range(16)]

            def body(v, c2):
                base = pl.multiple_of(v * 128, 128)
                for t in range(16):
                    wa = was[t]
                    wb = wbs[t]
                    for u in range(8):
                        sl = pl.ds(base + u * 16, 16)
                        o_v[t, sl] = wa * r0_v[t, sl] + wb * r1_v[t, sl]
                return c2

            lax.fori_loop(0, d // 128, body, 0)
            pltpu.sync_copy(o_v, out_hbm.at[pl.ds(w * 128 + j * 16, 16)])
            return carry

        lax.fori_loop(0, 8, chunk, 0)

    return _combine(bufout, sc0r, sc1r, w0r, w1r)


def kernel(x, gate_w, fc1_w, gating_w, fc2_w):
    bs, seq, d = x.shape
    e_num, hd, _ = fc1_w.shape
    n = bs * seq
    nk = n * _TOPK
    avg = (nk + e_num - 1) // e_num
    capacity = max(int(math.ceil(avg * 1.25)), 4)
    trash = e_num * capacity
    nrows = e_num * capacity

    chunk = 256
    n_chunks = n // chunk
    assert n % chunk == 0 and n % (_NW * 128) == 0

    x2d = x.reshape(n, d)

    router = pl.pallas_call(
        functools.partial(_router_body, capacity=capacity, e_num=e_num,
                          n_chunks=n_chunks, chunk=chunk, trash=trash),
        grid=(n_chunks,),
        in_specs=[
            pl.BlockSpec((chunk, d), lambda c: (c, 0)),
            pl.BlockSpec((e_num, d), lambda c: (0, 0)),
        ],
        out_specs=[
            pl.BlockSpec((chunk, 1), lambda c: (c, 0)),
            pl.BlockSpec((chunk, 1), lambda c: (c, 0)),
            pl.BlockSpec((chunk, 1), lambda c: (c, 0)),
            pl.BlockSpec((chunk, 1), lambda c: (c, 0)),
            pl.BlockSpec((chunk, 1), lambda c: (c, 0)),
            pl.BlockSpec((chunk, 1), lambda c: (c, 0)),
            pl.BlockSpec((1, e_num), lambda c: (0, 0)),
            pl.BlockSpec(memory_space=pltpu.SMEM),
        ],
        out_shape=[
            jax.ShapeDtypeStruct((n, 1), jnp.int32),
            jax.ShapeDtypeStruct((n, 1), jnp.int32),
            jax.ShapeDtypeStruct((n, 1), jnp.int32),
            jax.ShapeDtypeStruct((n, 1), jnp.int32),
            jax.ShapeDtypeStruct((n, 1), jnp.float32),
            jax.ShapeDtypeStruct((n, 1), jnp.float32),
            jax.ShapeDtypeStruct((1, e_num), jnp.float32),
            jax.ShapeDtypeStruct((1, 1), jnp.float32),
        ],
        scratch_shapes=[
            pltpu.VMEM((1, e_num), jnp.float32),
            pltpu.VMEM((1, e_num), jnp.float32),
        ],
    )
    sd0, sd1, sc0, sc1, w0, w1, counts, loss = router(x2d, gate_w)

    sd0r = sd0.reshape(n // 16, 16)
    sd1r = sd1.reshape(n // 16, 16)
    sc0r = sc0.reshape(n // 16, 16)
    sc1r = sc1.reshape(n // 16, 16)
    w0r = w0.reshape(n // 16, 16)
    w1r = w1.reshape(n // 16, 16)

    bufx = _dispatch_sc(x2d, sd0r, sd1r, nrows, d)

    hb = 1024
    nh = hd // hb
    ffn = pl.pallas_call(
        functools.partial(_ffn_body, nh=nh, rt=256, capacity=capacity),
        grid=(e_num, nh),
        in_specs=[
            pl.BlockSpec(memory_space=pltpu.SMEM),
            pl.BlockSpec((capacity, d), lambda e, h: (e, 0)),
            pl.BlockSpec((1, hb, d), lambda e, h: (e, h, 0)),
            pl.BlockSpec((1, hb, d), lambda e, h: (e, h, 0)),
            pl.BlockSpec((1, d, hb), lambda e, h: (e, 0, h)),
        ],
        out_specs=pl.BlockSpec((capacity, d), lambda e, h: (e, 0)),
        out_shape=jax.ShapeDtypeStruct((nrows, d), jnp.float32),
        scratch_shapes=[pltpu.VMEM((capacity, d), jnp.bfloat16)],
    )
    bufout = ffn(counts, bufx, fc1_w, gating_w, fc2_w)

    out = _combine_sc(bufout, sc0r, sc1r, w0r, w1r, n, d)

    return out.reshape(bs, seq, d).astype(x.dtype), loss[0, 0]


# speedup vs baseline: 1.8789x; 1.0344x over previous
import functools
import math

import jax
import jax.numpy as jnp
from jax import lax
from jax.experimental import pallas as pl
from jax.experimental.pallas import tpu as pltpu
from jax.experimental.pallas import tpu_sc as plsc

_TOPK = 2
_NC = 2
_NS = 16
_NW = _NC * _NS


def _router_body(x_ref, g_ref, sd0_ref, sd1_ref, sc0_ref, sc1_ref, w0_ref,
                 w1_ref, counts_ref, loss_ref, base_ref, psum_ref, *,
                 capacity, e_num, n_chunks, chunk, trash):
    c = pl.program_id(0)

    @pl.when(c == 0)
    def _():
        base_ref[...] = jnp.zeros_like(base_ref)
        psum_ref[...] = jnp.zeros_like(psum_ref)

    x = x_ref[...]
    g = g_ref[...]
    logits = lax.dot_general(x, g, (((1,), (1,)), ((), ())),
                             preferred_element_type=jnp.float32)
    m = jnp.max(logits, axis=1, keepdims=True)
    ex = jnp.exp(logits - m)
    p = ex / jnp.sum(ex, axis=1, keepdims=True)

    iota = lax.broadcasted_iota(jnp.int32, (chunk, e_num), 1)
    v1 = jnp.max(p, axis=1, keepdims=True)
    i1 = jnp.min(jnp.where(p == v1, iota, e_num), axis=1, keepdims=True)
    pm = jnp.where(iota == i1, -jnp.inf, p)
    v2 = jnp.max(pm, axis=1, keepdims=True)
    i2 = jnp.min(jnp.where(pm == v2, iota, e_num), axis=1, keepdims=True)
    s = v1 + v2 + 1e-9
    w0 = v1 / s
    w1 = v2 / s

    cmat = (iota == i1).astype(jnp.float32) + (iota == i2).astype(jnp.float32)

    ri = lax.broadcasted_iota(jnp.int32, (chunk, chunk), 0)
    ci = lax.broadcasted_iota(jnp.int32, (chunk, chunk), 1)
    ltri = (ci < ri).astype(jnp.float32)
    pref = base_ref[...] + lax.dot_general(
        ltri, cmat, (((1,), (0,)), ((), ())),
        preferred_element_type=jnp.float32)

    l0 = jnp.sum(jnp.where(iota == i1, pref, 0.0), axis=1,
                 keepdims=True).astype(jnp.int32)
    l1 = jnp.sum(jnp.where(iota == i2, pref, 0.0), axis=1,
                 keepdims=True).astype(jnp.int32)
    keep0 = l0 < capacity
    keep1 = l1 < capacity
    slot0 = i1 * capacity + l0
    slot1 = i2 * capacity + l1
    sd0_ref[...] = jnp.where(keep0, slot0, trash)
    sd1_ref[...] = jnp.where(keep1, slot1, trash)
    sc0_ref[...] = jnp.where(keep0, slot0, 0)
    sc1_ref[...] = jnp.where(keep1, slot1, 0)
    w0_ref[...] = jnp.where(keep0, w0, 0.0)
    w1_ref[...] = jnp.where(keep1, w1, 0.0)

    newbase = base_ref[...] + jnp.sum(cmat, axis=0, keepdims=True)
    base_ref[...] = newbase
    newpsum = psum_ref[...] + jnp.sum(p, axis=0, keepdims=True)
    psum_ref[...] = newpsum
    counts_ref[...] = newbase

    @pl.when(c == n_chunks - 1)
    def _():
        n_tok = n_chunks * chunk
        pmean = newpsum / float(n_tok)
        f = newbase / float(n_tok * _TOPK)
        loss_ref[0, 0] = jnp.sum(pmean * f) * float(e_num)


def _ffn_body(counts_ref, bufx_ref, w1_ref, wg_ref, w2_ref, out_ref, xb_ref,
              *, nh, rt, capacity):
    e = pl.program_id(0)
    h = pl.program_id(1)
    cnt = counts_ref[0, e].astype(jnp.int32)

    @pl.when(h == 0)
    def _():
        xb_ref[...] = bufx_ref[...].astype(jnp.bfloat16)

    w1b = w1_ref[0].astype(jnp.bfloat16)
    wgb = wg_ref[0].astype(jnp.bfloat16)
    w2b = w2_ref[0].astype(jnp.bfloat16)

    for i in range(capacity // rt):
        rs = pl.ds(i * rt, rt)

        @pl.when(i * rt < cnt)
        def _(i=i, rs=rs):
            x = xb_ref[rs, :]
            a = lax.dot_general(x, w1b, (((1,), (1,)), ((), ())),
                                preferred_element_type=jnp.float32)
            g = lax.dot_general(x, wgb, (((1,), (1,)), ((), ())),
                                preferred_element_type=jnp.float32)
            act = (a * jax.nn.sigmoid(a) * g).astype(jnp.bfloat16)
            delta = lax.dot_general(act, w2b, (((1,), (1,)), ((), ())),
                                    preferred_element_type=jnp.float32)

            @pl.when(h == 0)
            def _():
                out_ref[rs, :] = delta

            @pl.when(jnp.logical_and(h > 0, h < nh - 1))
            def _():
                out_ref[rs, :] = out_ref[rs, :] + delta

            @pl.when(h == nh - 1)
            def _(i=i):
                rows = i * rt + lax.broadcasted_iota(jnp.int32, (rt, 1), 0)
                acc = out_ref[rs, :] + delta
                out_ref[rs, :] = jnp.where(rows < cnt, acc, 0.0)

        @pl.when(jnp.logical_and(i * rt >= cnt, h == 0))
        def _(rs=rs):
            out_ref[rs, :] = jnp.zeros((rt, out_ref.shape[1]), jnp.float32)


def _dispatch_sc(x2d, sd0r, sd1r, nrows, d):
    mesh = plsc.VectorSubcoreMesh(core_axis_name="c", subcore_axis_name="s",
                                  num_cores=_NC)

    @functools.partial(
        pl.kernel,
        out_type=jax.ShapeDtypeStruct((nrows + 8, d), jnp.float32),
        mesh=mesh,
        scratch_types=[
            pltpu.VMEM((8, 16), jnp.int32),
            pltpu.VMEM((8, 16), jnp.int32),
            pltpu.VMEM((16, d), jnp.float32),
            pltpu.SemaphoreType.DMA,
        ],
    )
    def _dispatch(x_hbm, sd0_hbm, sd1_hbm, bufx_hbm, idx0_v, idx1_v, rows_v,
                  sem):
        w = lax.axis_index("s") * _NC + lax.axis_index("c")
        pltpu.sync_copy(sd0_hbm.at[pl.ds(w * 8, 8)], idx0_v)
        pltpu.sync_copy(sd1_hbm.at[pl.ds(w * 8, 8)], idx1_v)
        for j in range(8):
            pltpu.sync_copy(x_hbm.at[pl.ds(w * 128 + j * 16, 16)], rows_v)
            pltpu.async_copy(rows_v, bufx_hbm.at[idx0_v[j]], sem).wait()
            pltpu.async_copy(rows_v, bufx_hbm.at[idx1_v[j]], sem).wait()

    return _dispatch(x2d, sd0r, sd1r)


def _combine_sc(bufout, sc0r, sc1r, w0r, w1r, n, d):
    mesh = plsc.VectorSubcoreMesh(core_axis_name="c", subcore_axis_name="s",
                                  num_cores=_NC)

    @functools.partial(
        pl.kernel,
        out_type=jax.ShapeDtypeStruct((n, d), jnp.float32),
        mesh=mesh,
        scratch_types=[
            pltpu.VMEM((8, 16), jnp.int32),
            pltpu.VMEM((8, 16), jnp.int32),
            pltpu.VMEM((8, 16), jnp.float32),
            pltpu.VMEM((8, 16), jnp.float32),
            pltpu.VMEM((16, d), jnp.float32),
            pltpu.VMEM((16, d), jnp.float32),
            pltpu.VMEM((16, d), jnp.float32),
            pltpu.VMEM((16, d), jnp.float32),
            pltpu.VMEM((16, d), jnp.float32),
            pltpu.SemaphoreType.DMA,
            pltpu.SemaphoreType.DMA,
        ],
    )
    def _combine(bufout_hbm, sc0_hbm, sc1_hbm, w0_hbm, w1_hbm, out_hbm,
                 idx0_v, idx1_v, wa_v, wb_v, r0a_v, r1a_v, r0b_v, r1b_v,
                 o_v, sem_a, sem_b):
        w = lax.axis_index("s") * _NC + lax.axis_index("c")
        pltpu.sync_copy(sc0_hbm.at[pl.ds(w * 8, 8)], idx0_v)
        pltpu.sync_copy(sc1_hbm.at[pl.ds(w * 8, 8)], idx1_v)
        pltpu.sync_copy(w0_hbm.at[pl.ds(w * 8, 8)], wa_v)
        pltpu.sync_copy(w1_hbm.at[pl.ds(w * 8, 8)], wb_v)

        def compute(c, r0_v, r1_v):
            wav = wa_v[c]
            wbv = wb_v[c]
            was = [wav[t] for t in range(16)]
            wbs = [wbv[t] for t in range(16)]

            def body(v, c2):
                base = pl.multiple_of(v * 128, 128)
                for t in range(16):
                    wa = was[t]
                    wb = wbs[t]
                    for u in range(8):
                        sl = pl.ds(base + u * 16, 16)
                        o_v[t, sl] = wa * r0_v[t, sl] + wb * r1_v[t, sl]
                return c2

            lax.fori_loop(0, d // 128, body, 0)
            pltpu.sync_copy(o_v, out_hbm.at[pl.ds(w * 128 + c * 16, 16)])

        pltpu.async_copy(bufout_hbm.at[idx0_v[0]], r0a_v, sem_a)
        pltpu.async_copy(bufout_hbm.at[idx1_v[0]], r1a_v, sem_a)

        def pair(j, carry):
            c0 = 2 * j
            c1 = c0 + 1
            pltpu.async_copy(bufout_hbm.at[idx0_v[c1]], r0b_v, sem_b)
            pltpu.async_copy(bufout_hbm.at[idx1_v[c1]], r1b_v, sem_b)
            pltpu.make_async_copy(bufout_hbm.at[idx0_v[c0]], r0a_v,
                                  sem_a).wait()
            pltpu.make_async_copy(bufout_hbm.at[idx1_v[c0]], r1a_v,
                                  sem_a).wait()
            compute(c0, r0a_v, r1a_v)

            @pl.when(c1 + 1 < 8)
            def _():
                pltpu.async_copy(bufout_hbm.at[idx0_v[c1 + 1]], r0a_v, sem_a)
                pltpu.async_copy(bufout_hbm.at[idx1_v[c1 + 1]], r1a_v, sem_a)

            pltpu.make_async_copy(bufout_hbm.at[idx0_v[c1]], r0b_v,
                                  sem_b).wait()
            pltpu.make_async_copy(bufout_hbm.at[idx1_v[c1]], r1b_v,
                                  sem_b).wait()
            compute(c1, r0b_v, r1b_v)
            return carry

        lax.fori_loop(0, 4, pair, 0)

    return _combine(bufout, sc0r, sc1r, w0r, w1r)


def kernel(x, gate_w, fc1_w, gating_w, fc2_w):
    bs, seq, d = x.shape
    e_num, hd, _ = fc1_w.shape
    n = bs * seq
    nk = n * _TOPK
    avg = (nk + e_num - 1) // e_num
    capacity = max(int(math.ceil(avg * 1.25)), 4)
    trash = e_num * capacity
    nrows = e_num * capacity

    chunk = 256
    n_chunks = n // chunk
    assert n % chunk == 0 and n % (_NW * 128) == 0

    x2d = x.reshape(n, d)

    router = pl.pallas_call(
        functools.partial(_router_body, capacity=capacity, e_num=e_num,
                          n_chunks=n_chunks, chunk=chunk, trash=trash),
        grid=(n_chunks,),
        in_specs=[
            pl.BlockSpec((chunk, d), lambda c: (c, 0)),
            pl.BlockSpec((e_num, d), lambda c: (0, 0)),
        ],
        out_specs=[
            pl.BlockSpec((chunk, 1), lambda c: (c, 0)),
            pl.BlockSpec((chunk, 1), lambda c: (c, 0)),
            pl.BlockSpec((chunk, 1), lambda c: (c, 0)),
            pl.BlockSpec((chunk, 1), lambda c: (c, 0)),
            pl.BlockSpec((chunk, 1), lambda c: (c, 0)),
            pl.BlockSpec((chunk, 1), lambda c: (c, 0)),
            pl.BlockSpec((1, e_num), lambda c: (0, 0)),
            pl.BlockSpec(memory_space=pltpu.SMEM),
        ],
        out_shape=[
            jax.ShapeDtypeStruct((n, 1), jnp.int32),
            jax.ShapeDtypeStruct((n, 1), jnp.int32),
            jax.ShapeDtypeStruct((n, 1), jnp.int32),
            jax.ShapeDtypeStruct((n, 1), jnp.int32),
            jax.ShapeDtypeStruct((n, 1), jnp.float32),
            jax.ShapeDtypeStruct((n, 1), jnp.float32),
            jax.ShapeDtypeStruct((1, e_num), jnp.float32),
            jax.ShapeDtypeStruct((1, 1), jnp.float32),
        ],
        scratch_shapes=[
            pltpu.VMEM((1, e_num), jnp.float32),
            pltpu.VMEM((1, e_num), jnp.float32),
        ],
    )
    sd0, sd1, sc0, sc1, w0, w1, counts, loss = router(x2d, gate_w)

    sd0r = sd0.reshape(n // 16, 16)
    sd1r = sd1.reshape(n // 16, 16)
    sc0r = sc0.reshape(n // 16, 16)
    sc1r = sc1.reshape(n // 16, 16)
    w0r = w0.reshape(n // 16, 16)
    w1r = w1.reshape(n // 16, 16)

    bufx = _dispatch_sc(x2d, sd0r, sd1r, nrows, d)

    hb = 1024
    nh = hd // hb
    ffn = pl.pallas_call(
        functools.partial(_ffn_body, nh=nh, rt=256, capacity=capacity),
        grid=(e_num, nh),
        in_specs=[
            pl.BlockSpec(memory_space=pltpu.SMEM),
            pl.BlockSpec((capacity, d), lambda e, h: (e, 0)),
            pl.BlockSpec((1, hb, d), lambda e, h: (e, h, 0)),
            pl.BlockSpec((1, hb, d), lambda e, h: (e, h, 0)),
            pl.BlockSpec((1, d, hb), lambda e, h: (e, 0, h)),
        ],
        out_specs=pl.BlockSpec((capacity, d), lambda e, h: (e, 0)),
        out_shape=jax.ShapeDtypeStruct((nrows, d), jnp.float32),
        scratch_shapes=[pltpu.VMEM((capacity, d), jnp.bfloat16)],
    )
    bufout = ffn(counts, bufx, fc1_w, gating_w, fc2_w)

    out = _combine_sc(bufout, sc0r, sc1r, w0r, w1r, n, d)

    return out.reshape(bs, seq, d).astype(x.dtype), loss[0, 0]
